# baseline (speedup 1.0000x reference)
import jax
import jax.numpy as jnp
from jax import lax
from jax.experimental import pallas as pl
from jax.experimental.pallas import tpu as pltpu
from jax.experimental.pallas import tpu_sc as plsc

N = 10000
E = 160000
IN_DIM = 512
OUT_DIM = 256
EDGE_DIM = 16

QW = 64
ROWW = 64
NP = 10112
NPT = NP // 16
NOFF = (0, 128, 256, 384, NPT - 128)
CH = 80
EPT = CH * 128
EPAD = 16 * EPT
NEG_BIG = -1e30


def _tc_z_kernel(h_ref, wt_ref, wsd_ref, z_ref, s_ref):
    q = pl.program_id(0)
    zb = jnp.dot(h_ref[...], wt_ref[...], preferred_element_type=jnp.float32)
    quarter = jnp.where(
        q < 2,
        jnp.where(q == 0, zb[:, 0 * QW:1 * QW], zb[:, 1 * QW:2 * QW]),
        jnp.where(q == 2, zb[:, 2 * QW:3 * QW], zb[:, 3 * QW:4 * QW]),
    )
    z_ref[...] = quarter[None]
    s_ref[...] = jnp.dot(zb, wsd_ref[...], preferred_element_type=jnp.float32)


def _tc_t_kernel(ea_ref, we_ref, t_ref):
    t_ref[...] = jnp.dot(ea_ref[...], we_ref[...], preferred_element_type=jnp.float32)


def _lanemax_splat(x):
    lanes = lax.iota(jnp.int32, 16)
    dnums = lax.GatherDimensionNumbers(
        offset_dims=(), collapsed_slice_dims=(0,), start_index_map=(0,))
    for k in (1, 2, 4, 8):
        perm = jnp.bitwise_xor(lanes, k)
        shuf = lax.gather(x, perm[:, None], dnums, slice_sizes=(1,),
                          mode=lax.GatherScatterMode.PROMISE_IN_BOUNDS)
        x = jnp.maximum(x, shuf)
    return x


def _sc_kernel(srcF_hbm, dstF_hbm, tF_hbm, s1_hbm, s2_hbm, zq_hbm,
               out_hbm, src_f, dst_f, t_f, a1q, a2q, zb0, zb1, zb2, zb3,
               mx_v, num_sp, den_sp, maxb, sem0, sem1, sem2, sem3):
    c = lax.axis_index("c")
    s = lax.axis_index("s")
    base = s * NPT

    def zero_accum():
        def zrow(r, _):
            for k in range(ROWW // 16):
                zb0[r, pl.ds(k * 16, 16)] = jnp.zeros((16,), jnp.float32)
            return 0
        lax.fori_loop(0, 128, zrow, 0)
        for off in NOFF:
            pltpu.sync_copy(zb0, num_sp.at[pl.ds(base + off, 128)])

    zero_accum()
    def zq16(i, _):
        a1q[pl.ds(i * 16, 16)] = jnp.zeros((16,), jnp.float32)
        return 0
    lax.fori_loop(0, NPT // 16 + 1, zq16, 0)
    pltpu.sync_copy(a1q.at[pl.ds(0, NPT)], den_sp.at[pl.ds(base, NPT)])

    pltpu.sync_copy(srcF_hbm.at[s], src_f)
    pltpu.sync_copy(dstF_hbm.at[s], dst_f)
    pltpu.sync_copy(tF_hbm.at[s], t_f)

    coff = jnp.full((16,), c * NP, jnp.int32)
    QE = EPT // 4
    mrun = jnp.full((16,), -3e38, jnp.float32)
    for q in range(4):
        ga = pltpu.async_copy(
            s1_hbm.at[src_f.at[pl.ds(q * QE, QE)]], a1q, sem0)
        gb = pltpu.async_copy(
            s2_hbm.at[dst_f.at[pl.ds(q * QE, QE)]], a2q, sem1)
        ga.wait()
        gb.wait()

        def abody(i, mr, q=q):
            slq = pl.ds(i * 16, 16)
            sl = pl.ds(q * QE + i * 16, 16)
            ev = a1q[slq] + a2q[slq] + t_f[sl]
            ev = jnp.where(ev >= 0.0, ev, 0.01 * ev)
            t_f[sl] = ev
            mr = jnp.maximum(mr, ev)
            src_f[sl] = src_f[sl] + coff
            return mr
        mrun = lax.fori_loop(0, QE // 16, abody, mrun)

    a1q[pl.ds(0, 16)] = mrun
    pltpu.sync_copy(a1q.at[pl.ds(0, 16)], maxb.at[s])
    plsc.subcore_barrier()
    pltpu.sync_copy(maxb, mx_v)
    gv = mx_v[0, :]
    for k in range(1, 16):
        gv = jnp.maximum(gv, mx_v[k, :])
    gv = _lanemax_splat(gv)

    def cbody(i, _):
        sl = pl.ds(i * 16, 16)
        t_f[sl] = jnp.exp(t_f[sl] - gv)
        return 0
    lax.fori_loop(0, EPT // 16, cbody, 0)

    for p in range(2):
        if p == 1:
            step = jnp.full((16,), 2 * NP, jnp.int32)

            def obody(i, _):
                sl = pl.ds(i * 16, 16)
                src_f[sl] = src_f[sl] + step
                return 0
            lax.fori_loop(0, EPT // 16, obody, 0)
            zero_accum()

        plsc.subcore_barrier()

        def scale_scatter(buf, j):
            def sbody(q16, _):
                wvec = t_f[pl.ds(j * 128 + q16 * 16, 16)]
                for l in range(16):
                    r = q16 * 16 + l
                    wv = jnp.full((16,), wvec[l], jnp.float32)
                    for k in range(ROWW // 16):
                        sl = pl.ds(k * 16, 16)
                        buf[r, sl] = buf[r, sl] * wv
                return 0
            lax.fori_loop(0, 8, sbody, 0)
            pltpu.sync_copy(buf, num_sp.at[dst_f.at[pl.ds(j * 128, 128)]], add=True)

        def idx(j):
            return src_f.at[pl.ds(j * 128, 128)]

        bufs = (zb0, zb1, zb2, zb3)
        gsem = (sem0, sem1, sem2, sem3)
        for u in range(3):
            pltpu.async_copy(zq_hbm.at[idx(u)], bufs[u], gsem[u])

        if p == 0:
            def wsca(j8, _):
                pltpu.sync_copy(t_f.at[pl.ds(j8 * 1024, 1024)],
                                den_sp.at[dst_f.at[pl.ds(j8 * 1024, 1024)]],
                                add=True)
                return 0
            lax.fori_loop(0, EPT // 1024, wsca, 0)

        def dbody(j4, _):
            for u in range(4):
                m = 4 * j4 + u
                pltpu.make_async_copy(zq_hbm.at[idx(m)], bufs[u], gsem[u]).wait()
                scale_scatter(bufs[u], m)
                un = (u + 3) % 4
                if u == 0:
                    pltpu.async_copy(zq_hbm.at[idx(m + 3)], bufs[un], gsem[un])
                else:
                    @pl.when(m + 3 < CH)
                    def _():
                        pltpu.async_copy(zq_hbm.at[idx(m + 3)], bufs[un], gsem[un])
            return 0
        lax.fori_loop(0, CH // 4, dbody, 0)

        plsc.subcore_barrier()

        for off in NOFF:
            pltpu.sync_copy(num_sp.at[pl.ds(base + off, 128)], zb0)
            pltpu.sync_copy(den_sp.at[pl.ds(base + off, 128)],
                            a2q.at[pl.ds(0, 128)])

            def nbody(q16, _):
                dvec = a2q[pl.ds(q16 * 16, 16)]
                for l in range(16):
                    r = q16 * 16 + l
                    dv = jnp.maximum(jnp.full((16,), dvec[l], jnp.float32),
                                     1e-30)
                    for k in range(QW // 16):
                        sl = pl.ds(k * 16, 16)
                        zb0[r, sl] = zb0[r, sl] / dv
                return 0
            lax.fori_loop(0, 8, nbody, 0)
            pltpu.sync_copy(
                zb0, out_hbm.at[pl.ds((c + 2 * p) * NP + base + off, 128)])

        plsc.subcore_barrier()


@jax.jit
def _run(h, edge_index, edge_attr, W_fc, W_attn):
    src = edge_index[0].astype(jnp.int32)
    dst = edge_index[1].astype(jnp.int32)

    hp = jnp.concatenate([h, jnp.zeros((NP - N, IN_DIM), jnp.float32)], axis=0)
    Wt = W_fc.T
    w1 = W_attn[0, :OUT_DIM]
    w2 = W_attn[0, OUT_DIM + EDGE_DIM:]
    Wsd = jnp.stack([w1, w2] + [jnp.zeros_like(w1)] * 6, axis=1)

    RB = 632
    zq3, s8 = pl.pallas_call(
        _tc_z_kernel,
        grid=(4, NP // RB),
        in_specs=[
            pl.BlockSpec((RB, IN_DIM), lambda q, i: (i, 0)),
            pl.BlockSpec((IN_DIM, OUT_DIM), lambda q, i: (0, 0)),
            pl.BlockSpec((OUT_DIM, 8), lambda q, i: (0, 0)),
        ],
        out_specs=[
            pl.BlockSpec((1, RB, ROWW), lambda q, i: (q, i, 0)),
            pl.BlockSpec((RB, 8), lambda q, i: (i, 0)),
        ],
        out_shape=[
            jax.ShapeDtypeStruct((4, NP, ROWW), jnp.float32),
            jax.ShapeDtypeStruct((NP, 8), jnp.float32),
        ],
    )(hp, Wt, Wsd)
    zq = zq3.reshape(4 * NP, ROWW)
    s1 = s8[:, 0]
    s2 = s8[:, 1]

    eap = jnp.concatenate(
        [edge_attr, jnp.zeros((EPAD - E, EDGE_DIM), jnp.float32)], axis=0)
    we = W_attn[0, OUT_DIM:OUT_DIM + EDGE_DIM]
    We8 = jnp.stack([we] + [jnp.zeros_like(we)] * 7, axis=1)
    EB = 2048
    t8 = pl.pallas_call(
        _tc_t_kernel,
        grid=(EPAD // EB,),
        in_specs=[
            pl.BlockSpec((EB, EDGE_DIM), lambda i: (i, 0)),
            pl.BlockSpec((EDGE_DIM, 8), lambda i: (0, 0)),
        ],
        out_specs=pl.BlockSpec((EB, 8), lambda i: (i, 0)),
        out_shape=jax.ShapeDtypeStruct((EPAD, 8), jnp.float32),
    )(eap, We8)

    pad = EPAD - E
    EPT_ = CH * 128
    srcf = jnp.concatenate([src, jnp.zeros((pad,), jnp.int32)]).reshape(16, EPT_)
    dstf = jnp.concatenate([dst, jnp.zeros((pad,), jnp.int32)]).reshape(16, EPT_)
    tf = jnp.concatenate(
        [t8[:E, 0], jnp.full((pad,), NEG_BIG, jnp.float32)]).reshape(16, EPT_)

    mesh = plsc.VectorSubcoreMesh(core_axis_name="c", subcore_axis_name="s")
    sc = pl.kernel(
        _sc_kernel,
        out_type=jax.ShapeDtypeStruct((4 * NP, ROWW), jnp.float32),
        mesh=mesh,
        compiler_params=pltpu.CompilerParams(
            use_tc_tiling_on_sc=False, needs_layout_passes=False),
        scratch_types=[
            pltpu.VMEM((EPT,), jnp.int32),
            pltpu.VMEM((EPT,), jnp.int32),
            pltpu.VMEM((EPT,), jnp.float32),
            pltpu.VMEM((EPT // 4,), jnp.float32),
            pltpu.VMEM((EPT // 4,), jnp.float32),
            pltpu.VMEM((128, ROWW), jnp.float32),
            pltpu.VMEM((128, ROWW), jnp.float32),
            pltpu.VMEM((128, ROWW), jnp.float32),
            pltpu.VMEM((128, ROWW), jnp.float32),
            pltpu.VMEM((16, 16), jnp.float32),
            pltpu.VMEM_SHARED((NP, ROWW), jnp.float32),
            pltpu.VMEM_SHARED((NP,), jnp.float32),
            pltpu.VMEM_SHARED((16, 16), jnp.float32),
            pltpu.SemaphoreType.DMA,
            pltpu.SemaphoreType.DMA,
            pltpu.SemaphoreType.DMA,
            pltpu.SemaphoreType.DMA,
        ],
    )
    outq = sc(srcf, dstf, tf, s1, s2, zq)

    return jnp.concatenate(
        [outq[0 * NP:0 * NP + N], outq[1 * NP:1 * NP + N],
         outq[2 * NP:2 * NP + N], outq[3 * NP:3 * NP + N]], axis=1)


def kernel(h, edge_index, edge_attr, W_fc, W_attn):
    return _run(h, edge_index, edge_attr, W_fc, W_attn)


# speedup vs baseline: 1.1674x; 1.1674x over previous
import jax
import jax.numpy as jnp
from jax import lax
from jax.experimental import pallas as pl
from jax.experimental.pallas import tpu as pltpu
from jax.experimental.pallas import tpu_sc as plsc

N = 10000
E = 160000
IN_DIM = 512
OUT_DIM = 256
EDGE_DIM = 16

QW = 64
ROWW = 64
NP = 10112
NPT = NP // 16
NOFF = (0, 128, 256, 384, NPT - 128)
CH = 80
EPT = CH * 128
EPAD = 16 * EPT
NEG_BIG = -1e30


def _tc_z_kernel(h_ref, wt_ref, wsd_ref, z_ref, s_ref):
    q = pl.program_id(0)
    zb = jnp.dot(h_ref[...], wt_ref[...], preferred_element_type=jnp.float32)
    quarter = jnp.where(
        q < 2,
        jnp.where(q == 0, zb[:, 0 * QW:1 * QW], zb[:, 1 * QW:2 * QW]),
        jnp.where(q == 2, zb[:, 2 * QW:3 * QW], zb[:, 3 * QW:4 * QW]),
    )
    z_ref[...] = quarter[None]
    s_ref[...] = jnp.dot(zb, wsd_ref[...], preferred_element_type=jnp.float32)


def _tc_t_kernel(ea_ref, we_ref, t_ref):
    t_ref[...] = jnp.dot(ea_ref[...], we_ref[...], preferred_element_type=jnp.float32)


def _lanemax_splat(x):
    lanes = lax.iota(jnp.int32, 16)
    dnums = lax.GatherDimensionNumbers(
        offset_dims=(), collapsed_slice_dims=(0,), start_index_map=(0,))
    for k in (1, 2, 4, 8):
        perm = jnp.bitwise_xor(lanes, k)
        shuf = lax.gather(x, perm[:, None], dnums, slice_sizes=(1,),
                          mode=lax.GatherScatterMode.PROMISE_IN_BOUNDS)
        x = jnp.maximum(x, shuf)
    return x


def _sc_kernel(srcF_hbm, dstF_hbm, tF_hbm, s1_hbm, s2_hbm, zq_hbm,
               out_hbm, src_f, dst_f, t_f, a1q, a2q, zb0, zb1, zb2, zb3,
               mx_v, den_l, den_idx, num_sp, den_sp, maxb,
               sem0, sem1, sem2, sem3):
    c = lax.axis_index("c")
    s = lax.axis_index("s")
    base = s * NPT

    def zero_accum():
        def zrow(r, _):
            for k in range(ROWW // 16):
                zb0[r, pl.ds(k * 16, 16)] = jnp.zeros((16,), jnp.float32)
            return 0
        lax.fori_loop(0, 128, zrow, 0)
        for off in NOFF:
            pltpu.sync_copy(zb0, num_sp.at[pl.ds(base + off, 128)])

    zero_accum()
    def zq16(i, _):
        a1q[pl.ds(i * 16, 16)] = jnp.zeros((16,), jnp.float32)
        return 0
    lax.fori_loop(0, NPT // 16 + 1, zq16, 0)
    pltpu.sync_copy(a1q.at[pl.ds(0, NPT)], den_sp.at[pl.ds(base, NPT)])
    lanes16 = lax.iota(jnp.int32, 16)

    def zdl(i, _):
        sl = pl.ds(i * 16, 16)
        den_l[sl] = jnp.zeros((16,), jnp.float32)
        den_idx[sl] = lanes16 + jnp.full((16,), i * 16, jnp.int32)
        return 0
    lax.fori_loop(0, NP // 16, zdl, 0)

    pltpu.sync_copy(srcF_hbm.at[s], src_f)
    pltpu.sync_copy(dstF_hbm.at[s], dst_f)
    pltpu.sync_copy(tF_hbm.at[s], t_f)

    coff = jnp.full((16,), c * NP, jnp.int32)
    QE = EPT // 4
    mrun = jnp.full((16,), -3e38, jnp.float32)
    for q in range(4):
        ga = pltpu.async_copy(
            s1_hbm.at[src_f.at[pl.ds(q * QE, QE)]], a1q, sem0)
        gb = pltpu.async_copy(
            s2_hbm.at[dst_f.at[pl.ds(q * QE, QE)]], a2q, sem1)
        ga.wait()
        gb.wait()

        def abody(i, mr, q=q):
            slq = pl.ds(i * 16, 16)
            sl = pl.ds(q * QE + i * 16, 16)
            ev = a1q[slq] + a2q[slq] + t_f[sl]
            ev = jnp.where(ev >= 0.0, ev, 0.01 * ev)
            t_f[sl] = ev
            mr = jnp.maximum(mr, ev)
            src_f[sl] = src_f[sl] + coff
            return mr
        mrun = lax.fori_loop(0, QE // 16, abody, mrun)

    a1q[pl.ds(0, 16)] = mrun
    pltpu.sync_copy(a1q.at[pl.ds(0, 16)], maxb.at[s])
    plsc.subcore_barrier()
    pltpu.sync_copy(maxb, mx_v)
    gv = mx_v[0, :]
    for k in range(1, 16):
        gv = jnp.maximum(gv, mx_v[k, :])
    gv = _lanemax_splat(gv)

    def cbody(i, _):
        sl = pl.ds(i * 16, 16)
        w = jnp.exp(t_f[sl] - gv)
        t_f[sl] = w
        plsc.addupdate_scatter(den_l, [dst_f[sl]], w)
        return 0
    lax.fori_loop(0, EPT // 16, cbody, 0)
    pltpu.sync_copy(den_l, den_sp.at[den_idx], add=True)

    for p in range(2):
        if p == 1:
            step = jnp.full((16,), 2 * NP, jnp.int32)

            def obody(i, _):
                sl = pl.ds(i * 16, 16)
                src_f[sl] = src_f[sl] + step
                return 0
            lax.fori_loop(0, EPT // 16, obody, 0)
            zero_accum()

        plsc.subcore_barrier()

        def scale_scatter(buf, j):
            def sbody(q16, _):
                wvec = t_f[pl.ds(j * 128 + q16 * 16, 16)]
                for l in range(16):
                    r = q16 * 16 + l
                    wv = jnp.full((16,), wvec[l], jnp.float32)
                    for k in range(ROWW // 16):
                        sl = pl.ds(k * 16, 16)
                        buf[r, sl] = buf[r, sl] * wv
                return 0
            lax.fori_loop(0, 8, sbody, 0)
            pltpu.sync_copy(buf, num_sp.at[dst_f.at[pl.ds(j * 128, 128)]], add=True)

        def idx(j):
            return src_f.at[pl.ds(j * 128, 128)]

        bufs = (zb0, zb1, zb2, zb3)
        gsem = (sem0, sem1, sem2, sem3)
        for u in range(3):
            pltpu.async_copy(zq_hbm.at[idx(u)], bufs[u], gsem[u])

        def dbody(j4, _):
            for u in range(4):
                m = 4 * j4 + u
                pltpu.make_async_copy(zq_hbm.at[idx(m)], bufs[u], gsem[u]).wait()
                scale_scatter(bufs[u], m)
                un = (u + 3) % 4
                if u == 0:
                    pltpu.async_copy(zq_hbm.at[idx(m + 3)], bufs[un], gsem[un])
                else:
                    @pl.when(m + 3 < CH)
                    def _():
                        pltpu.async_copy(zq_hbm.at[idx(m + 3)], bufs[un], gsem[un])
            return 0
        lax.fori_loop(0, CH // 4, dbody, 0)

        plsc.subcore_barrier()

        for off in NOFF:
            pltpu.sync_copy(num_sp.at[pl.ds(base + off, 128)], zb0)
            pltpu.sync_copy(den_sp.at[pl.ds(base + off, 128)],
                            a2q.at[pl.ds(0, 128)])

            def nbody(q16, _):
                dvec = a2q[pl.ds(q16 * 16, 16)]
                for l in range(16):
                    r = q16 * 16 + l
                    dv = jnp.maximum(jnp.full((16,), dvec[l], jnp.float32),
                                     1e-30)
                    for k in range(QW // 16):
                        sl = pl.ds(k * 16, 16)
                        zb0[r, sl] = zb0[r, sl] / dv
                return 0
            lax.fori_loop(0, 8, nbody, 0)
            pltpu.sync_copy(
                zb0, out_hbm.at[pl.ds((c + 2 * p) * NP + base + off, 128)])

        plsc.subcore_barrier()


@jax.jit
def _run(h, edge_index, edge_attr, W_fc, W_attn):
    src = edge_index[0].astype(jnp.int32)
    dst = edge_index[1].astype(jnp.int32)

    hp = jnp.concatenate([h, jnp.zeros((NP - N, IN_DIM), jnp.float32)], axis=0)
    Wt = W_fc.T
    w1 = W_attn[0, :OUT_DIM]
    w2 = W_attn[0, OUT_DIM + EDGE_DIM:]
    Wsd = jnp.stack([w1, w2] + [jnp.zeros_like(w1)] * 6, axis=1)

    RB = 632
    zq3, s8 = pl.pallas_call(
        _tc_z_kernel,
        grid=(4, NP // RB),
        in_specs=[
            pl.BlockSpec((RB, IN_DIM), lambda q, i: (i, 0)),
            pl.BlockSpec((IN_DIM, OUT_DIM), lambda q, i: (0, 0)),
            pl.BlockSpec((OUT_DIM, 8), lambda q, i: (0, 0)),
        ],
        out_specs=[
            pl.BlockSpec((1, RB, ROWW), lambda q, i: (q, i, 0)),
            pl.BlockSpec((RB, 8), lambda q, i: (i, 0)),
        ],
        out_shape=[
            jax.ShapeDtypeStruct((4, NP, ROWW), jnp.float32),
            jax.ShapeDtypeStruct((NP, 8), jnp.float32),
        ],
    )(hp, Wt, Wsd)
    zq = zq3.reshape(4 * NP, ROWW)
    s1 = s8[:, 0]
    s2 = s8[:, 1]

    eap = jnp.concatenate(
        [edge_attr, jnp.zeros((EPAD - E, EDGE_DIM), jnp.float32)], axis=0)
    we = W_attn[0, OUT_DIM:OUT_DIM + EDGE_DIM]
    We8 = jnp.stack([we] + [jnp.zeros_like(we)] * 7, axis=1)
    EB = 2048
    t8 = pl.pallas_call(
        _tc_t_kernel,
        grid=(EPAD // EB,),
        in_specs=[
            pl.BlockSpec((EB, EDGE_DIM), lambda i: (i, 0)),
            pl.BlockSpec((EDGE_DIM, 8), lambda i: (0, 0)),
        ],
        out_specs=pl.BlockSpec((EB, 8), lambda i: (i, 0)),
        out_shape=jax.ShapeDtypeStruct((EPAD, 8), jnp.float32),
    )(eap, We8)

    pad = EPAD - E
    EPT_ = CH * 128
    srcf = jnp.concatenate([src, jnp.zeros((pad,), jnp.int32)]).reshape(16, EPT_)
    dstf = jnp.concatenate([dst, jnp.zeros((pad,), jnp.int32)]).reshape(16, EPT_)
    tf = jnp.concatenate(
        [t8[:E, 0], jnp.full((pad,), NEG_BIG, jnp.float32)]).reshape(16, EPT_)

    mesh = plsc.VectorSubcoreMesh(core_axis_name="c", subcore_axis_name="s")
    sc = pl.kernel(
        _sc_kernel,
        out_type=jax.ShapeDtypeStruct((4 * NP, ROWW), jnp.float32),
        mesh=mesh,
        compiler_params=pltpu.CompilerParams(
            use_tc_tiling_on_sc=False, needs_layout_passes=False),
        scratch_types=[
            pltpu.VMEM((EPT,), jnp.int32),
            pltpu.VMEM((EPT,), jnp.int32),
            pltpu.VMEM((EPT,), jnp.float32),
            pltpu.VMEM((EPT // 4,), jnp.float32),
            pltpu.VMEM((EPT // 4,), jnp.float32),
            pltpu.VMEM((128, ROWW), jnp.float32),
            pltpu.VMEM((128, ROWW), jnp.float32),
            pltpu.VMEM((128, ROWW), jnp.float32),
            pltpu.VMEM((128, ROWW), jnp.float32),
            pltpu.VMEM((16, 16), jnp.float32),
            pltpu.VMEM((NP,), jnp.float32),
            pltpu.VMEM((NP,), jnp.int32),
            pltpu.VMEM_SHARED((NP, ROWW), jnp.float32),
            pltpu.VMEM_SHARED((NP,), jnp.float32),
            pltpu.VMEM_SHARED((16, 16), jnp.float32),
            pltpu.SemaphoreType.DMA,
            pltpu.SemaphoreType.DMA,
            pltpu.SemaphoreType.DMA,
            pltpu.SemaphoreType.DMA,
        ],
    )
    outq = sc(srcf, dstf, tf, s1, s2, zq)

    return jnp.concatenate(
        [outq[0 * NP:0 * NP + N], outq[1 * NP:1 * NP + N],
         outq[2 * NP:2 * NP + N], outq[3 * NP:3 * NP + N]], axis=1)


def kernel(h, edge_index, edge_attr, W_fc, W_attn):
    return _run(h, edge_index, edge_attr, W_fc, W_attn)
